# Initial kernel scaffold; baseline (speedup 1.0000x reference)
import jax
import jax.numpy as jnp
from jax.experimental import pallas as pl


def kernel(doc_w, doc_c, qry_w, qry_c, k_layer, K, W, char_table, conv_w, conv_b):
    raise NotImplementedError("write your pallas kernel here")


# speedup vs baseline: 8.6057x; 8.6057x over previous
import functools

import jax
import jax.numpy as jnp
from jax import lax
from jax.experimental import pallas as pl
from jax.experimental.pallas import tpu as pltpu
from jax.experimental.pallas import tpu_sc as plsc

VOCAB = 100000
EMB = 128
NCHAR = 128
CDIM = 16
FSIZE = 64
FWIDTH = 5
B = 64
DL = 512
QL = 32
WL = 16
NPOS = WL - FWIDTH + 1
OUT = EMB + FSIZE

NW = 32
ND = B * DL
NQ = B * QL

DW_ROWS = ND // NW // 128
DTOK = ND // NW
QTOK = NQ // NW
CCHUNK = 128


def _sc_gather(Wt, ctT, dw, qw, dc, qc):
    mesh = plsc.VectorSubcoreMesh(core_axis_name="c", subcore_axis_name="s")

    @functools.partial(
        pl.kernel,
        mesh=mesh,
        compiler_params=pltpu.CompilerParams(needs_layout_passes=False),
        out_type=[
            jax.ShapeDtypeStruct((ND, EMB), jnp.float32),
            jax.ShapeDtypeStruct((NQ, EMB), jnp.float32),
            jax.ShapeDtypeStruct((ND, WL * CDIM), jnp.float32),
            jax.ShapeDtypeStruct((NQ, WL * CDIM), jnp.float32),
        ],
        scratch_types=[
            pltpu.VMEM((16, 128), jnp.int32),
            pltpu.VMEM((512, EMB), jnp.float32),
            pltpu.VMEM((CDIM, NCHAR), jnp.float32),
            pltpu.VMEM((CCHUNK * WL,), jnp.int32),
            pltpu.VMEM((CCHUNK, WL * CDIM), jnp.float32),
            pltpu.SemaphoreType.DMA,
        ],
    )
    def k(w_hbm, ctT_hbm, dw_hbm, qw_hbm, dc_hbm, qc_hbm,
          wd_out, wq_out, cd_out, cq_out,
          idx_v, wrows, ctT_v, ids_v, cemb_v, sem):
        wid = lax.axis_index("s") * 2 + lax.axis_index("c")

        for s in range(2):
            pltpu.sync_copy(dw_hbm.at[pl.ds(wid * DW_ROWS + s * 4, 4)],
                            idx_v.at[pl.ds(0, 4)])
            cps = [pltpu.async_copy(w_hbm.at[idx_v.at[jj]],
                                    wrows.at[pl.ds(jj * 128, 128)], sem)
                   for jj in range(4)]
            for cp in cps:
                cp.wait()
            pltpu.sync_copy(wrows, wd_out.at[pl.ds(wid * DTOK + s * 512, 512)])

        @pl.when(wid < 16)
        def _():
            pltpu.sync_copy(qw_hbm.at[wid], idx_v.at[0])
            pltpu.async_copy(w_hbm.at[idx_v.at[0]],
                             wrows.at[pl.ds(0, 128)], sem).wait()
            pltpu.sync_copy(wrows.at[pl.ds(0, 128)],
                            wq_out.at[pl.ds(wid * 128, 128)])

        pltpu.sync_copy(ctT_hbm, ctT_v)

        def char_chunk(ids_hbm, out_hbm, tok_off, ntok):
            pltpu.sync_copy(ids_hbm.at[pl.ds(tok_off * WL, ntok * WL)],
                            ids_v.at[pl.ds(0, ntok * WL)])

            def tok_body(j, c):
                ids = ids_v[pl.ds(j * WL, WL)]
                for d in range(CDIM):
                    dvec = jnp.full((WL,), d, jnp.int32)
                    cemb_v[j, pl.ds(d * WL, WL)] = plsc.load_gather(
                        ctT_v, [dvec, ids])
                return c

            lax.fori_loop(0, ntok, tok_body, 0)
            pltpu.sync_copy(cemb_v.at[pl.ds(0, ntok)],
                            out_hbm.at[pl.ds(tok_off, ntok)])

        def dc_body(s, carry):
            char_chunk(dc_hbm, cd_out, wid * DTOK + s * CCHUNK, CCHUNK)
            return carry

        lax.fori_loop(0, DTOK // CCHUNK, dc_body, 0)
        char_chunk(qc_hbm, cq_out, wid * QTOK, QTOK)

    return k(Wt, ctT, dw, qw, dc, qc)


def _tc_conv(x, wemb, M, bias, n, blk):
    def body(x_ref, w_ref, m_ref, b_ref, o_ref):
        y = jnp.dot(x_ref[...], m_ref[...],
                    preferred_element_type=jnp.float32)
        acc = y[:, 0:FSIZE]
        for p in range(1, NPOS):
            acc = jnp.maximum(acc, y[:, p * FSIZE:(p + 1) * FSIZE])
        acc = jnp.maximum(acc + b_ref[...], 0.0)
        o_ref[:, 0:EMB] = w_ref[...]
        o_ref[:, EMB:OUT] = acc

    return pl.pallas_call(
        body,
        grid=(n // blk,),
        in_specs=[
            pl.BlockSpec((blk, WL * CDIM), lambda i: (i, 0)),
            pl.BlockSpec((blk, EMB), lambda i: (i, 0)),
            pl.BlockSpec((WL * CDIM, NPOS * FSIZE), lambda i: (0, 0)),
            pl.BlockSpec((1, FSIZE), lambda i: (0, 0)),
        ],
        out_specs=pl.BlockSpec((blk, OUT), lambda i: (i, 0)),
        out_shape=jax.ShapeDtypeStruct((n, OUT), jnp.float32),
    )(x, wemb, M, bias)


def _build_band(conv_w):
    wct = jnp.transpose(conv_w[:, :, 0, :], (2, 1, 0))
    m4 = jnp.zeros((WL, CDIM, NPOS, FSIZE), jnp.float32)
    for p in range(NPOS):
        m4 = m4.at[p:p + FWIDTH, :, p, :].set(wct)
    return jnp.transpose(m4, (1, 0, 2, 3)).reshape(WL * CDIM, NPOS * FSIZE)


def kernel(doc_w, doc_c, qry_w, qry_c, k_layer, K, W, char_table, conv_w, conv_b):
    dw = doc_w.astype(jnp.int32).reshape(ND // 128, 128)
    qw = qry_w.astype(jnp.int32).reshape(NQ // 128, 128)
    dc = doc_c.astype(jnp.int32).reshape(ND * WL)
    qc = qry_c.astype(jnp.int32).reshape(NQ * WL)
    Wt = W.astype(jnp.float32)
    ctT = char_table.astype(jnp.float32).T

    wd, wq, cd, cq = _sc_gather(Wt, ctT, dw, qw, dc, qc)

    M = _build_band(conv_w.astype(jnp.float32))
    bias = conv_b.astype(jnp.float32).reshape(1, FSIZE)

    outd = _tc_conv(cd, wd, M, bias, ND, 1024)
    outq = _tc_conv(cq, wq, M, bias, NQ, 512)
    return outd.reshape(B, DL, OUT), outq.reshape(B, QL, OUT)
